# baseline (speedup 1.0000x reference)
import functools

import jax
import jax.numpy as jnp
from jax import lax
from jax.experimental import pallas as pl
from jax.experimental.pallas import tpu as pltpu
from jax.experimental.pallas import tpu_sc as plsc

NAME_DIM = 32
HIDDEN = 64
NUM_WORKERS = 32
LANES = 16
CHUNK = 1024
TC_R = 2048


def _sc_transpose(table_t):
  nvocab = table_t.shape[1]
  full_windows = nvocab // 128
  tail = nvocab % 128
  base, extra = divmod(full_windows, NUM_WORKERS)

  mesh = plsc.VectorSubcoreMesh(core_axis_name="c", subcore_axis_name="s")

  @functools.partial(
      pl.kernel,
      mesh=mesh,
      out_type=jax.ShapeDtypeStruct((nvocab // 4, 128), jnp.float32),
      scratch_types=[
          pltpu.VMEM((8, 512), jnp.float32),
          pltpu.VMEM((32, 128), jnp.float32),
          pltpu.SemaphoreType.DMA,
      ],
      compiler_params=pltpu.CompilerParams(needs_layout_passes=False),
  )
  def transpose_kernel(tt_hbm, out_hbm, stage_v, out_v, sem):
    wid = lax.axis_index("s") * 2 + lax.axis_index("c")
    lo = wid * base + jnp.minimum(wid, extra)
    hi = lo + base + jnp.where(wid < extra, 1, 0)

    iota16 = lax.iota(jnp.int32, LANES)
    rows16 = iota16 & 7
    half128 = lax.shift_left(lax.shift_right_logical(iota16, 3), 7)

    def compose_rows(nrows):
      def row_body(r, carry2):
        for g in range(8):
          cols16 = half128 + (128 * 2) * (g % 2) + 4 * r + (g // 2)
          vals = plsc.load_gather(stage_v, [rows16, cols16])
          out_v[r, pl.ds(16 * g, LANES)] = vals
        return carry2

      lax.fori_loop(0, nrows, row_body, 0)

    def window_body(j, carry):
      col0 = j * 128
      copies = [
          pltpu.async_copy(
              tt_hbm.at[pl.ds(8 * fb, 8), pl.ds(col0, 128)],
              stage_v.at[:, pl.ds(128 * fb, 128)],
              sem,
          )
          for fb in range(4)
      ]
      for cp in copies:
        cp.wait()
      compose_rows(32)
      row0 = pl.multiple_of(col0 // 4, 8)
      pltpu.sync_copy(out_v, out_hbm.at[pl.ds(row0, 32), :])
      return carry

    lax.fori_loop(lo, hi, window_body, 0)

  return transpose_kernel(table_t)


def _sc_gather(name_table, idx2d, vals2d, zeros128, n):
  rows_per_worker = n // NUM_WORKERS
  chunks = rows_per_worker // CHUNK

  mesh = plsc.VectorSubcoreMesh(core_axis_name="c", subcore_axis_name="s")

  @functools.partial(
      pl.kernel,
      mesh=mesh,
      out_type=(
          jax.ShapeDtypeStruct((n, NAME_DIM), jnp.float32),
          jax.ShapeDtypeStruct((n // 4, 128), jnp.float32),
      ),
      scratch_types=[
          pltpu.VMEM((8, 128), jnp.int32),
          pltpu.VMEM((8, 128), jnp.float32),
          pltpu.VMEM((CHUNK, NAME_DIM), jnp.float32),
          pltpu.VMEM((CHUNK // 4, 128), jnp.float32),
          pltpu.SemaphoreType.DMA,
      ],
      compiler_params=pltpu.CompilerParams(
          use_tc_tiling_on_sc=False, needs_layout_passes=False),
  )
  def gather_kernel(table_hbm, idx_hbm, val_hbm, z_hbm, out_hbm, vi_hbm,
                    idx_v, val_v, rows_v, vi_v, sem):
    wid = lax.axis_index("s") * 2 + lax.axis_index("c")
    idx_row0 = wid * (chunks * 8)
    out0 = wid * rows_per_worker

    pltpu.sync_copy(z_hbm, vi_v)

    iota16 = lax.iota(jnp.int32, LANES)
    rows_base = lax.shift_right_logical(iota16, 2)
    lanes_v = lax.shift_left(iota16 & 3, 5)
    lanes_1 = lanes_v + 1
    ones16 = jnp.ones((LANES,), jnp.float32)

    def chunk_body(i, carry):
      r0 = idx_row0 + i * 8
      tok0 = out0 + i * CHUNK
      pltpu.sync_copy(idx_hbm.at[pl.ds(r0, 8), :], idx_v)
      copies = [
          pltpu.async_copy(
              table_hbm.at[idx_v.at[j]],
              rows_v.at[pl.ds(j * 128, 128), :],
              sem,
          )
          for j in range(8)
      ]
      pltpu.sync_copy(val_hbm.at[pl.ds(r0, 8), :], val_v)

      def group_body(g, carry2):
        v16 = val_v[g // 8, pl.ds((g % 8) * LANES, LANES)]
        rows16 = rows_base + g * 4
        plsc.store_scatter(vi_v, [rows16, lanes_v], v16)
        plsc.store_scatter(vi_v, [rows16, lanes_1], ones16)
        return carry2

      lax.fori_loop(0, CHUNK // LANES, group_body, 0)
      pltpu.sync_copy(vi_v, vi_hbm.at[pl.ds(tok0 // 4, CHUNK // 4), :])
      for cp in copies:
        cp.wait()
      pltpu.sync_copy(rows_v, out_hbm.at[pl.ds(tok0, CHUNK), :])
      return carry

    lax.fori_loop(0, chunks, chunk_body, 0)

  return gather_kernel(name_table, idx2d, vals2d, zeros128)


def _tc_body(g_ref, vi_ref, wa_ref, wv_ref, out_ref):
  i = pl.program_id(0)
  m = jnp.dot(g_ref[...], wa_ref[...], preferred_element_type=jnp.float32)
  m = m + jnp.dot(vi_ref[...], wv_ref[...], preferred_element_type=jnp.float32)
  h = jnp.maximum(m, 0.0)
  p = jnp.dot(jnp.ones((1, TC_R), jnp.float32), h,
              preferred_element_type=jnp.float32)

  @pl.when(i == 0)
  def _():
    out_ref[...] = jnp.zeros_like(out_ref)

  out_ref[...] += p

  @pl.when(i == pl.num_programs(0) - 1)
  def _():
    out_ref[...] *= (1.0 / (pl.num_programs(0) * TC_R * 4))


def kernel(test_names, test_values, name_table, W_val, b_val, W_fc, b_fc):
  n = test_names.shape[0]
  idx2d = test_names.reshape(n // 128, 128)
  vals2d = test_values.reshape(n // 128, 128)
  zeros128 = jnp.zeros((CHUNK // 4, 128), jnp.float32)
  packed = _sc_transpose(name_table.T)
  nvocab = name_table.shape[0]
  tail_start = (nvocab // 128) * 128
  tail16 = name_table[tail_start:].reshape((nvocab - tail_start) // 4, 128)
  packed = lax.dynamic_update_slice(packed, tail16, (tail_start // 4, 0))
  table_lin = packed.reshape(nvocab, NAME_DIM)
  gathered, vi4 = _sc_gather(table_lin, idx2d, vals2d, zeros128, n)
  g4 = gathered.reshape(n // 4, 128)

  a = W_fc[:NAME_DIM]
  tail = W_fc[NAME_DIM:]
  u = W_val[0] @ tail
  c = b_val @ tail + b_fc
  wa = jnp.zeros((128, 256), jnp.float32)
  wv = jnp.zeros((128, 256), jnp.float32)
  for q in range(4):
    wa = wa.at[32 * q:32 * q + NAME_DIM, 64 * q:64 * q + HIDDEN].set(a)
    wv = wv.at[32 * q, 64 * q:64 * q + HIDDEN].set(u)
    wv = wv.at[32 * q + 1, 64 * q:64 * q + HIDDEN].set(c)

  grid = (n // 4) // TC_R
  out = pl.pallas_call(
      _tc_body,
      grid=(grid,),
      in_specs=[
          pl.BlockSpec((TC_R, 128), lambda i: (i, 0)),
          pl.BlockSpec((TC_R, 128), lambda i: (i, 0)),
          pl.BlockSpec((128, 256), lambda i: (0, 0)),
          pl.BlockSpec((128, 256), lambda i: (0, 0)),
      ],
      out_specs=pl.BlockSpec((1, 256), lambda i: (0, 0)),
      out_shape=jax.ShapeDtypeStruct((1, 256), jnp.float32),
  )(g4, vi4, wa, wv)
  return out[0].reshape(4, HIDDEN).sum(axis=0)


# speedup vs baseline: 1.1347x; 1.1347x over previous
import functools

import jax
import jax.numpy as jnp
from jax import lax
from jax.experimental import pallas as pl
from jax.experimental.pallas import tpu as pltpu
from jax.experimental.pallas import tpu_sc as plsc

NAME_DIM = 32
HIDDEN = 64
NUM_WORKERS = 32
LANES = 16
CHUNK = 1024
TC_R = 2048


def _sc_transpose(table_t):
  nvocab = table_t.shape[1]
  full_windows = nvocab // 128
  tail = nvocab % 128
  base, extra = divmod(full_windows, NUM_WORKERS)

  groups = full_windows // 4
  gbase, gextra = divmod(groups, NUM_WORKERS)

  mesh = plsc.VectorSubcoreMesh(core_axis_name="c", subcore_axis_name="s")

  @functools.partial(
      pl.kernel,
      mesh=mesh,
      out_type=jax.ShapeDtypeStruct((nvocab // 4, 128), jnp.float32),
      scratch_types=[
          pltpu.VMEM((8, 2048), jnp.float32),
          pltpu.VMEM((128, 128), jnp.float32),
          pltpu.SemaphoreType.DMA,
      ],
      compiler_params=pltpu.CompilerParams(needs_layout_passes=False),
  )
  def transpose_kernel(tt_hbm, out_hbm, stage_v, out_v, sem):
    wid = lax.axis_index("s") * 2 + lax.axis_index("c")
    lo = wid * gbase + jnp.minimum(wid, gextra)
    hi = lo + gbase + jnp.where(wid < gextra, 1, 0)

    iota16 = lax.iota(jnp.int32, LANES)
    rows16 = iota16 & 7
    half512 = lax.shift_left(lax.shift_right_logical(iota16, 3), 9)

    def group_body(g, carry):
      col0 = g * 512
      copies = [
          pltpu.async_copy(
              tt_hbm.at[pl.ds(8 * fb, 8), pl.ds(col0, 512)],
              stage_v.at[:, pl.ds(512 * fb, 512)],
              sem,
          )
          for fb in range(4)
      ]
      @pl.when(g > lo)
      def _():
        row0 = pl.multiple_of((g - 1) * 128, 8)
        pltpu.sync_copy(out_v, out_hbm.at[pl.ds(row0, 128), :])

      for cp in copies:
        cp.wait()
      for sub in range(4):
        def row_body(r, carry2, _sub=sub):
          for g8 in range(8):
            k = 1024 * (g8 % 2) + 128 * _sub + (g8 // 2)
            cols16 = half512 + (k + 4 * r)
            vals = plsc.load_gather(stage_v, [rows16, cols16])
            out_v[32 * _sub + r, pl.ds(16 * g8, LANES)] = vals
          return carry2

        lax.fori_loop(0, 32, row_body, 0)
      return carry

    lax.fori_loop(lo, hi, group_body, 0)
    row_last = pl.multiple_of((hi - 1) * 128, 8)
    pltpu.sync_copy(out_v, out_hbm.at[pl.ds(row_last, 128), :])

  return transpose_kernel(table_t)


def _sc_gather(name_table, idx2d, vals2d, zeros128, n):
  rows_per_worker = n // NUM_WORKERS
  chunks = rows_per_worker // CHUNK

  mesh = plsc.VectorSubcoreMesh(core_axis_name="c", subcore_axis_name="s")

  @functools.partial(
      pl.kernel,
      mesh=mesh,
      out_type=(
          jax.ShapeDtypeStruct((n, NAME_DIM), jnp.float32),
          jax.ShapeDtypeStruct((n // 4, 128), jnp.float32),
      ),
      scratch_types=[
          pltpu.VMEM((8, 128), jnp.int32),
          pltpu.VMEM((8, 128), jnp.float32),
          pltpu.VMEM((CHUNK, NAME_DIM), jnp.float32),
          pltpu.VMEM((CHUNK // 4, 128), jnp.float32),
          pltpu.SemaphoreType.DMA,
      ],
      compiler_params=pltpu.CompilerParams(
          use_tc_tiling_on_sc=False, needs_layout_passes=False),
  )
  def gather_kernel(table_hbm, idx_hbm, val_hbm, z_hbm, out_hbm, vi_hbm,
                    idx_v, val_v, rows_v, vi_v, sem):
    wid = lax.axis_index("s") * 2 + lax.axis_index("c")
    idx_row0 = wid * (chunks * 8)
    out0 = wid * rows_per_worker

    pltpu.sync_copy(z_hbm, vi_v)

    iota16 = lax.iota(jnp.int32, LANES)
    rows_base = lax.shift_right_logical(iota16, 2)
    lanes_v = lax.shift_left(iota16 & 3, 5)
    lanes_1 = lanes_v + 1
    ones16 = jnp.ones((LANES,), jnp.float32)

    def chunk_body(i, carry):
      r0 = idx_row0 + i * 8
      tok0 = out0 + i * CHUNK
      pltpu.sync_copy(idx_hbm.at[pl.ds(r0, 8), :], idx_v)
      copies = [
          pltpu.async_copy(
              table_hbm.at[idx_v.at[j]],
              rows_v.at[pl.ds(j * 128, 128), :],
              sem,
          )
          for j in range(8)
      ]
      pltpu.sync_copy(val_hbm.at[pl.ds(r0, 8), :], val_v)

      def group_body(g, carry2):
        v16 = val_v[g // 8, pl.ds((g % 8) * LANES, LANES)]
        rows16 = rows_base + g * 4
        plsc.store_scatter(vi_v, [rows16, lanes_v], v16)
        plsc.store_scatter(vi_v, [rows16, lanes_1], ones16)
        return carry2

      lax.fori_loop(0, CHUNK // LANES, group_body, 0)
      pltpu.sync_copy(vi_v, vi_hbm.at[pl.ds(tok0 // 4, CHUNK // 4), :])
      for cp in copies:
        cp.wait()
      pltpu.sync_copy(rows_v, out_hbm.at[pl.ds(tok0, CHUNK), :])
      return carry

    lax.fori_loop(0, chunks, chunk_body, 0)

  return gather_kernel(name_table, idx2d, vals2d, zeros128)


def _tc_body(g_ref, vi_ref, wa_ref, wv_ref, out_ref):
  i = pl.program_id(0)
  m = jnp.dot(g_ref[...], wa_ref[...], preferred_element_type=jnp.float32)
  m = m + jnp.dot(vi_ref[...], wv_ref[...], preferred_element_type=jnp.float32)
  h = jnp.maximum(m, 0.0)
  p = jnp.dot(jnp.ones((1, TC_R), jnp.float32), h,
              preferred_element_type=jnp.float32)

  @pl.when(i == 0)
  def _():
    out_ref[...] = jnp.zeros_like(out_ref)

  out_ref[...] += p

  @pl.when(i == pl.num_programs(0) - 1)
  def _():
    out_ref[...] *= (1.0 / (pl.num_programs(0) * TC_R * 4))


def kernel(test_names, test_values, name_table, W_val, b_val, W_fc, b_fc):
  n = test_names.shape[0]
  idx2d = test_names.reshape(n // 128, 128)
  vals2d = test_values.reshape(n // 128, 128)
  zeros128 = jnp.zeros((CHUNK // 4, 128), jnp.float32)
  packed = _sc_transpose(name_table.T)
  nvocab = name_table.shape[0]
  tail_start = (nvocab // 128) * 128
  tail16 = name_table[tail_start:].reshape((nvocab - tail_start) // 4, 128)
  packed = lax.dynamic_update_slice(packed, tail16, (tail_start // 4, 0))
  table_lin = packed.reshape(nvocab, NAME_DIM)
  gathered, vi4 = _sc_gather(table_lin, idx2d, vals2d, zeros128, n)
  g4 = gathered.reshape(n // 4, 128)

  a = W_fc[:NAME_DIM]
  tail = W_fc[NAME_DIM:]
  u = W_val[0] @ tail
  c = b_val @ tail + b_fc
  wa = jnp.zeros((128, 256), jnp.float32)
  wv = jnp.zeros((128, 256), jnp.float32)
  for q in range(4):
    wa = wa.at[32 * q:32 * q + NAME_DIM, 64 * q:64 * q + HIDDEN].set(a)
    wv = wv.at[32 * q, 64 * q:64 * q + HIDDEN].set(u)
    wv = wv.at[32 * q + 1, 64 * q:64 * q + HIDDEN].set(c)

  grid = (n // 4) // TC_R
  out = pl.pallas_call(
      _tc_body,
      grid=(grid,),
      in_specs=[
          pl.BlockSpec((TC_R, 128), lambda i: (i, 0)),
          pl.BlockSpec((TC_R, 128), lambda i: (i, 0)),
          pl.BlockSpec((128, 256), lambda i: (0, 0)),
          pl.BlockSpec((128, 256), lambda i: (0, 0)),
      ],
      out_specs=pl.BlockSpec((1, 256), lambda i: (0, 0)),
      out_shape=jax.ShapeDtypeStruct((1, 256), jnp.float32),
  )(g4, vi4, wa, wv)
  return out[0].reshape(4, HIDDEN).sum(axis=0)


# speedup vs baseline: 3.2327x; 2.8490x over previous
import functools

import jax
import jax.numpy as jnp
from jax import lax
from jax.experimental import pallas as pl
from jax.experimental.pallas import tpu as pltpu
from jax.experimental.pallas import tpu_sc as plsc

NAME_DIM = 32
HIDDEN = 64
NUM_WORKERS = 32
LANES = 16
CHUNK = 1024
TC_R = 2048


def _sc_transpose(table_t):
  nvocab = table_t.shape[1]
  full_windows = nvocab // 128
  tail = nvocab % 128
  base, extra = divmod(full_windows, NUM_WORKERS)

  groups = full_windows // 4
  gbase, gextra = divmod(groups, NUM_WORKERS)

  mesh = plsc.VectorSubcoreMesh(core_axis_name="c", subcore_axis_name="s")

  @functools.partial(
      pl.kernel,
      mesh=mesh,
      out_type=jax.ShapeDtypeStruct((nvocab // 4, 128), jnp.float32),
      scratch_types=[
          pltpu.VMEM((8, 2048), jnp.float32),
          pltpu.VMEM((128, 128), jnp.float32),
          pltpu.SemaphoreType.DMA,
      ],
      compiler_params=pltpu.CompilerParams(needs_layout_passes=False),
  )
  def transpose_kernel(tt_hbm, out_hbm, stage_v, out_v, sem):
    wid = lax.axis_index("s") * 2 + lax.axis_index("c")
    lo = wid * gbase + jnp.minimum(wid, gextra)
    hi = lo + gbase + jnp.where(wid < gextra, 1, 0)

    iota16 = lax.iota(jnp.int32, LANES)
    rows16 = iota16 & 7
    half512 = lax.shift_left(lax.shift_right_logical(iota16, 3), 9)

    def group_body(g, carry):
      col0 = g * 512
      copies = [
          pltpu.async_copy(
              tt_hbm.at[pl.ds(8 * fb, 8), pl.ds(col0, 512)],
              stage_v.at[:, pl.ds(512 * fb, 512)],
              sem,
          )
          for fb in range(4)
      ]
      @pl.when(g > lo)
      def _():
        row0 = pl.multiple_of((g - 1) * 128, 8)
        pltpu.sync_copy(out_v, out_hbm.at[pl.ds(row0, 128), :])

      for cp in copies:
        cp.wait()
      for sub in range(4):
        @functools.partial(plsc.parallel_loop, 0, 32, unroll=4)
        def _(r, _sub=sub):
          for g8 in range(8):
            k = 1024 * (g8 % 2) + 128 * _sub + (g8 // 2)
            cols16 = half512 + (k + 4 * r)
            vals = plsc.load_gather(stage_v, [rows16, cols16])
            out_v[32 * _sub + r, pl.ds(16 * g8, LANES)] = vals
      return carry

    lax.fori_loop(lo, hi, group_body, 0)
    row_last = pl.multiple_of((hi - 1) * 128, 8)
    pltpu.sync_copy(out_v, out_hbm.at[pl.ds(row_last, 128), :])

  return transpose_kernel(table_t)


def _sc_gather(name_table, idx2d, vals2d, zeros128, n):
  rows_per_worker = n // NUM_WORKERS
  chunks = rows_per_worker // CHUNK

  mesh = plsc.VectorSubcoreMesh(core_axis_name="c", subcore_axis_name="s")

  @functools.partial(
      pl.kernel,
      mesh=mesh,
      out_type=(
          jax.ShapeDtypeStruct((n, NAME_DIM), jnp.float32),
          jax.ShapeDtypeStruct((n // 4, 128), jnp.float32),
      ),
      scratch_types=[
          pltpu.VMEM((8, 128), jnp.int32),
          pltpu.VMEM((8, 128), jnp.float32),
          pltpu.VMEM((CHUNK, NAME_DIM), jnp.float32),
          pltpu.VMEM((CHUNK // 4, 128), jnp.float32),
          pltpu.SemaphoreType.DMA,
      ],
      compiler_params=pltpu.CompilerParams(
          use_tc_tiling_on_sc=False, needs_layout_passes=False),
  )
  def gather_kernel(table_hbm, idx_hbm, val_hbm, z_hbm, out_hbm, vi_hbm,
                    idx_v, val_v, rows_v, vi_v, sem):
    wid = lax.axis_index("s") * 2 + lax.axis_index("c")
    idx_row0 = wid * (chunks * 8)
    out0 = wid * rows_per_worker

    pltpu.sync_copy(z_hbm, vi_v)

    iota16 = lax.iota(jnp.int32, LANES)
    rows_base = lax.shift_right_logical(iota16, 2)
    lanes_v = lax.shift_left(iota16 & 3, 5)
    lanes_1 = lanes_v + 1
    ones16 = jnp.ones((LANES,), jnp.float32)

    def chunk_body(i, carry):
      r0 = idx_row0 + i * 8
      tok0 = out0 + i * CHUNK
      pltpu.sync_copy(idx_hbm.at[pl.ds(r0, 8), :], idx_v)
      copies = [
          pltpu.async_copy(
              table_hbm.at[idx_v.at[j]],
              rows_v.at[pl.ds(j * 128, 128), :],
              sem,
          )
          for j in range(8)
      ]
      pltpu.sync_copy(val_hbm.at[pl.ds(r0, 8), :], val_v)

      def group_body(g, carry2):
        v16 = val_v[g // 8, pl.ds((g % 8) * LANES, LANES)]
        rows16 = rows_base + g * 4
        plsc.store_scatter(vi_v, [rows16, lanes_v], v16)
        plsc.store_scatter(vi_v, [rows16, lanes_1], ones16)
        return carry2

      lax.fori_loop(0, CHUNK // LANES, group_body, 0)
      pltpu.sync_copy(vi_v, vi_hbm.at[pl.ds(tok0 // 4, CHUNK // 4), :])
      for cp in copies:
        cp.wait()
      pltpu.sync_copy(rows_v, out_hbm.at[pl.ds(tok0, CHUNK), :])
      return carry

    lax.fori_loop(0, chunks, chunk_body, 0)

  return gather_kernel(name_table, idx2d, vals2d, zeros128)


def _tc_body(g_ref, vi_ref, wa_ref, wv_ref, out_ref):
  i = pl.program_id(0)
  m = jnp.dot(g_ref[...], wa_ref[...], preferred_element_type=jnp.float32)
  m = m + jnp.dot(vi_ref[...], wv_ref[...], preferred_element_type=jnp.float32)
  h = jnp.maximum(m, 0.0)
  p = jnp.dot(jnp.ones((1, TC_R), jnp.float32), h,
              preferred_element_type=jnp.float32)

  @pl.when(i == 0)
  def _():
    out_ref[...] = jnp.zeros_like(out_ref)

  out_ref[...] += p

  @pl.when(i == pl.num_programs(0) - 1)
  def _():
    out_ref[...] *= (1.0 / (pl.num_programs(0) * TC_R * 4))


def kernel(test_names, test_values, name_table, W_val, b_val, W_fc, b_fc):
  n = test_names.shape[0]
  idx2d = test_names.reshape(n // 128, 128)
  vals2d = test_values.reshape(n // 128, 128)
  zeros128 = jnp.zeros((CHUNK // 4, 128), jnp.float32)
  packed = _sc_transpose(name_table.T)
  nvocab = name_table.shape[0]
  tail_start = (nvocab // 128) * 128
  tail16 = name_table[tail_start:].reshape((nvocab - tail_start) // 4, 128)
  packed = lax.dynamic_update_slice(packed, tail16, (tail_start // 4, 0))
  table_lin = packed.reshape(nvocab, NAME_DIM)
  gathered, vi4 = _sc_gather(table_lin, idx2d, vals2d, zeros128, n)
  g4 = gathered.reshape(n // 4, 128)

  a = W_fc[:NAME_DIM]
  tail = W_fc[NAME_DIM:]
  u = W_val[0] @ tail
  c = b_val @ tail + b_fc
  wa = jnp.zeros((128, 256), jnp.float32)
  wv = jnp.zeros((128, 256), jnp.float32)
  for q in range(4):
    wa = wa.at[32 * q:32 * q + NAME_DIM, 64 * q:64 * q + HIDDEN].set(a)
    wv = wv.at[32 * q, 64 * q:64 * q + HIDDEN].set(u)
    wv = wv.at[32 * q + 1, 64 * q:64 * q + HIDDEN].set(c)

  grid = (n // 4) // TC_R
  out = pl.pallas_call(
      _tc_body,
      grid=(grid,),
      in_specs=[
          pl.BlockSpec((TC_R, 128), lambda i: (i, 0)),
          pl.BlockSpec((TC_R, 128), lambda i: (i, 0)),
          pl.BlockSpec((128, 256), lambda i: (0, 0)),
          pl.BlockSpec((128, 256), lambda i: (0, 0)),
      ],
      out_specs=pl.BlockSpec((1, 256), lambda i: (0, 0)),
      out_shape=jax.ShapeDtypeStruct((1, 256), jnp.float32),
  )(g4, vi4, wa, wv)
  return out[0].reshape(4, HIDDEN).sum(axis=0)
